# baseline (speedup 1.0000x reference)
import jax
import jax.numpy as jnp
from jax import lax
from jax.experimental import pallas as pl
from jax.experimental.pallas import tpu as pltpu
from jax.experimental.pallas import tpu_sc as plsc

B, C, H, W = 4, 96, 512, 512
N = B * H * W
NW = 32
PPW = N // NW
BLK = 4096
NBLK = PPW // BLK
GPB = BLK // 16
NDMA = 4
IPD = BLK // NDMA


def _body(z_hbm, l_hbm, lt_hbm, cond_hbm, out_hbm,
          l_v0, l_v1, lt_v0, lt_v1, c_v0, c_v1,
          idxl_v0, idxl_v1, idxlt_v0, idxlt_v1,
          good_v0, good_v1, bad_v0, bad_v1, acc_v,
          sem_in0, sem_in1, sem_g0, sem_g1):
    wid = lax.axis_index("s") * 2 + lax.axis_index("c")
    base = wid * PPW
    bC = (wid // 8) * C
    lanes = lax.iota(jnp.int32, 16)
    l_v = [l_v0, l_v1]
    lt_v = [lt_v0, lt_v1]
    c_v = [c_v0, c_v1]
    idxl_v = [idxl_v0, idxl_v1]
    idxlt_v = [idxlt_v0, idxlt_v1]
    good_v = [good_v0, good_v1]
    bad_v = [bad_v0, bad_v1]
    sem_in = [sem_in0, sem_in1]
    sem_g = [sem_g0, sem_g1]

    def issue_inputs(i):
        s = i % 2
        sl = pl.ds(base + i * BLK, BLK)
        return [pltpu.async_copy(l_hbm.at[sl], l_v[s], sem_in[s]),
                pltpu.async_copy(lt_hbm.at[sl], lt_v[s], sem_in[s]),
                pltpu.async_copy(cond_hbm.at[sl], c_v[s], sem_in[s])]

    def compute_idx_and_fire(i):
        s = i % 2
        p0 = base + i * BLK
        cps = []
        for k in range(NDMA):
            g0 = k * (IPD // 16)

            def mkidx(g, _):
                geo = ((p0 + g * 16) & 262143) + lanes
                sl = pl.ds(g * 16, 16)
                lv = l_v[s][sl]
                ltv = lt_v[s][sl]
                idxl_v[s][sl] = ((bC + lv) << 18) + geo
                idxlt_v[s][sl] = ((bC + ltv) << 18) + geo
                return 0

            lax.fori_loop(g0, g0 + IPD // 16, mkidx, 0)
            sl = pl.ds(k * IPD, IPD)
            cps.append(pltpu.async_copy(
                z_hbm.at[idxl_v[s].at[sl]], good_v[s].at[sl], sem_g[s]))
            cps.append(pltpu.async_copy(
                z_hbm.at[idxlt_v[s].at[sl]], bad_v[s].at[sl], sem_g[s]))
        return cps

    def accum(i, acc):
        s = i % 2

        def body(g, a):
            sl = pl.ds(g * 16, 16)
            return a + (good_v[s][sl] - bad_v[s][sl]) * c_v[s][sl]

        return lax.fori_loop(0, GPB, body, acc)

    acc = jnp.zeros((16,), jnp.float32)
    in_cps = issue_inputs(0)
    gath_prev = None
    for i in range(NBLK):
        for cp in in_cps:
            cp.wait()
        gath_cur = compute_idx_and_fire(i)
        if gath_prev is not None:
            for cp in gath_prev:
                cp.wait()
            acc = accum(i - 1, acc)
        in_cps = issue_inputs(i + 1) if i + 1 < NBLK else []
        gath_prev = gath_cur
    for cp in gath_prev:
        cp.wait()
    acc = accum(NBLK - 1, acc)

    acc_v[...] = acc
    pltpu.sync_copy(acc_v, out_hbm.at[wid])


def _phys_view(x):
    s = x.shape
    return (x.reshape(*s[:-2], s[-2] // 8, 8, s[-1] // 128, 128)
            .swapaxes(-2, -3)
            .reshape(-1))


def kernel(z, condition, l, l_target):
    z_phys = _phys_view(z)
    l_phys = _phys_view(l.astype(jnp.int32))
    lt_phys = _phys_view(l_target.astype(jnp.int32))
    cond_phys = _phys_view(condition.astype(jnp.float32))

    mesh = plsc.VectorSubcoreMesh(core_axis_name="c", subcore_axis_name="s")
    fn = pl.kernel(
        _body,
        mesh=mesh,
        out_type=jax.ShapeDtypeStruct((NW, 16), jnp.float32),
        scratch_types=(
            [pltpu.VMEM((BLK,), jnp.int32)] * 2 +
            [pltpu.VMEM((BLK,), jnp.int32)] * 2 +
            [pltpu.VMEM((BLK,), jnp.float32)] * 2 +
            [pltpu.VMEM((BLK,), jnp.int32)] * 2 +
            [pltpu.VMEM((BLK,), jnp.int32)] * 2 +
            [pltpu.VMEM((BLK,), jnp.float32)] * 2 +
            [pltpu.VMEM((BLK,), jnp.float32)] * 2 +
            [pltpu.VMEM((16,), jnp.float32)] +
            [pltpu.SemaphoreType.DMA] * 4
        ),
    )
    partials = fn(z_phys, l_phys, lt_phys, cond_phys)
    return jnp.sum(partials)


# speedup vs baseline: 1.0046x; 1.0046x over previous
import jax
import jax.numpy as jnp
from jax import lax
from jax.experimental import pallas as pl
from jax.experimental.pallas import tpu as pltpu
from jax.experimental.pallas import tpu_sc as plsc

B, C, H, W = 4, 96, 512, 512
N = B * H * W
NW = 32
PPW = N // NW
BLK = 8192
NBLK = PPW // BLK
GPB = BLK // 16
NDMA = 4
IPD = BLK // NDMA


def _body(z_hbm, l_hbm, lt_hbm, cond_hbm, out_hbm,
          l_v0, l_v1, lt_v0, lt_v1, c_v0, c_v1,
          idxl_v0, idxl_v1, idxlt_v0, idxlt_v1,
          good_v0, good_v1, bad_v0, bad_v1, acc_v,
          sem_in0, sem_in1, sem_g0, sem_g1):
    wid = lax.axis_index("s") * 2 + lax.axis_index("c")
    base = wid * PPW
    bC = (wid // 8) * C
    lanes = lax.iota(jnp.int32, 16)
    l_v = [l_v0, l_v1]
    lt_v = [lt_v0, lt_v1]
    c_v = [c_v0, c_v1]
    idxl_v = [idxl_v0, idxl_v1]
    idxlt_v = [idxlt_v0, idxlt_v1]
    good_v = [good_v0, good_v1]
    bad_v = [bad_v0, bad_v1]
    sem_in = [sem_in0, sem_in1]
    sem_g = [sem_g0, sem_g1]

    def issue_inputs(i):
        s = i % 2
        sl = pl.ds(base + i * BLK, BLK)
        return [pltpu.async_copy(l_hbm.at[sl], l_v[s], sem_in[s]),
                pltpu.async_copy(lt_hbm.at[sl], lt_v[s], sem_in[s]),
                pltpu.async_copy(cond_hbm.at[sl], c_v[s], sem_in[s])]

    def compute_idx_and_fire(i):
        s = i % 2
        p0 = base + i * BLK
        cps = []
        for k in range(NDMA):
            g0 = k * (IPD // 16)

            def mkidx(g, _):
                geo = ((p0 + g * 16) & 262143) + lanes
                sl = pl.ds(g * 16, 16)
                lv = l_v[s][sl]
                ltv = lt_v[s][sl]
                idxl_v[s][sl] = ((bC + lv) << 18) + geo
                idxlt_v[s][sl] = ((bC + ltv) << 18) + geo
                return 0

            lax.fori_loop(g0, g0 + IPD // 16, mkidx, 0)
            sl = pl.ds(k * IPD, IPD)
            cps.append(pltpu.async_copy(
                z_hbm.at[idxl_v[s].at[sl]], good_v[s].at[sl], sem_g[s]))
            cps.append(pltpu.async_copy(
                z_hbm.at[idxlt_v[s].at[sl]], bad_v[s].at[sl], sem_g[s]))
        return cps

    def accum(i, acc):
        s = i % 2

        def body(g, a):
            sl = pl.ds(g * 16, 16)
            return a + (good_v[s][sl] - bad_v[s][sl]) * c_v[s][sl]

        return lax.fori_loop(0, GPB, body, acc)

    acc = jnp.zeros((16,), jnp.float32)
    in_cps = issue_inputs(0)
    gath_prev = None
    for i in range(NBLK):
        for cp in in_cps:
            cp.wait()
        gath_cur = compute_idx_and_fire(i)
        if gath_prev is not None:
            for cp in gath_prev:
                cp.wait()
            acc = accum(i - 1, acc)
        in_cps = issue_inputs(i + 1) if i + 1 < NBLK else []
        gath_prev = gath_cur
    for cp in gath_prev:
        cp.wait()
    acc = accum(NBLK - 1, acc)

    acc_v[...] = acc
    pltpu.sync_copy(acc_v, out_hbm.at[wid])


def _phys_view(x):
    s = x.shape
    return (x.reshape(*s[:-2], s[-2] // 8, 8, s[-1] // 128, 128)
            .swapaxes(-2, -3)
            .reshape(-1))


def kernel(z, condition, l, l_target):
    z_phys = _phys_view(z)
    l_phys = _phys_view(l.astype(jnp.int32))
    lt_phys = _phys_view(l_target.astype(jnp.int32))
    cond_phys = _phys_view(condition.astype(jnp.float32))

    mesh = plsc.VectorSubcoreMesh(core_axis_name="c", subcore_axis_name="s")
    fn = pl.kernel(
        _body,
        mesh=mesh,
        out_type=jax.ShapeDtypeStruct((NW, 16), jnp.float32),
        scratch_types=(
            [pltpu.VMEM((BLK,), jnp.int32)] * 2 +
            [pltpu.VMEM((BLK,), jnp.int32)] * 2 +
            [pltpu.VMEM((BLK,), jnp.float32)] * 2 +
            [pltpu.VMEM((BLK,), jnp.int32)] * 2 +
            [pltpu.VMEM((BLK,), jnp.int32)] * 2 +
            [pltpu.VMEM((BLK,), jnp.float32)] * 2 +
            [pltpu.VMEM((BLK,), jnp.float32)] * 2 +
            [pltpu.VMEM((16,), jnp.float32)] +
            [pltpu.SemaphoreType.DMA] * 4
        ),
    )
    partials = fn(z_phys, l_phys, lt_phys, cond_phys)
    return jnp.sum(partials)


# speedup vs baseline: 1.0078x; 1.0032x over previous
import jax
import jax.numpy as jnp
from jax import lax
from jax.experimental import pallas as pl
from jax.experimental.pallas import tpu as pltpu
from jax.experimental.pallas import tpu_sc as plsc

B, C, H, W = 4, 96, 512, 512
N = B * H * W
NW = 32
PPW = N // NW
BLK = 8192
NBLK = PPW // BLK
GPB = BLK // 16
NDMA = 4
IPD = BLK // NDMA


def _body(z_hbm, l_hbm, lt_hbm, cond_hbm, out_hbm,
          l_v0, l_v1, lt_v0, lt_v1, c_v0, c_v1,
          idxl_v0, idxl_v1, idxlt_v0, idxlt_v1,
          good_v0, good_v1, bad_v0, bad_v1, acc_v,
          sem_in0, sem_in1, sem_g0, sem_g1):
    wid = lax.axis_index("s") * 2 + lax.axis_index("c")
    base = wid * PPW
    bC = (wid // 8) * C
    lanes = lax.iota(jnp.int32, 16)
    l_v = [l_v0, l_v1]
    lt_v = [lt_v0, lt_v1]
    c_v = [c_v0, c_v1]
    idxl_v = [idxl_v0, idxl_v1]
    idxlt_v = [idxlt_v0, idxlt_v1]
    good_v = [good_v0, good_v1]
    bad_v = [bad_v0, bad_v1]
    sem_in = [sem_in0, sem_in1]
    sem_g = [sem_g0, sem_g1]

    def issue_inputs(i):
        s = i % 2
        sl = pl.ds(base + i * BLK, BLK)
        return [pltpu.async_copy(l_hbm.at[sl], l_v[s], sem_in[s]),
                pltpu.async_copy(lt_hbm.at[sl], lt_v[s], sem_in[s]),
                pltpu.async_copy(cond_hbm.at[sl], c_v[s], sem_in[s])]

    def compute_idx_and_fire(i):
        s = i % 2
        p0 = base + i * BLK
        cps = []
        for k in range(NDMA):
            g0 = k * (IPD // 16)

            def mkidx(g, _):
                geo = ((p0 + g * 16) & 262143) + lanes
                sl = pl.ds(g * 16, 16)
                lv = l_v[s][sl]
                ltv = lt_v[s][sl]
                idxl_v[s][sl] = ((bC + lv) << 18) + geo
                idxlt_v[s][sl] = ((bC + ltv) << 18) + geo
                return 0

            lax.fori_loop(g0, g0 + IPD // 16, mkidx, 0)
            sl = pl.ds(k * IPD, IPD)
            cps.append(pltpu.async_copy(
                z_hbm.at[idxl_v[s].at[sl]], good_v[s].at[sl], sem_g[s]))
            cps.append(pltpu.async_copy(
                z_hbm.at[idxlt_v[s].at[sl]], bad_v[s].at[sl], sem_g[s]))
        return cps

    def accum_chunk(i, k, acc):
        s = i % 2
        g0 = k * (IPD // 16)

        def body(g, a):
            sl = pl.ds(g * 16, 16)
            return a + (good_v[s][sl] - bad_v[s][sl]) * c_v[s][sl]

        return lax.fori_loop(g0, g0 + IPD // 16, body, acc)

    def drain_and_accum(i, cps, acc):
        for k in range(NDMA):
            cps[2 * k].wait()
            cps[2 * k + 1].wait()
            acc = accum_chunk(i, k, acc)
        return acc

    acc = jnp.zeros((16,), jnp.float32)
    in_cps = issue_inputs(0)
    gath_prev = None
    for i in range(NBLK):
        for cp in in_cps:
            cp.wait()
        gath_cur = compute_idx_and_fire(i)
        if gath_prev is not None:
            acc = drain_and_accum(i - 1, gath_prev, acc)
        in_cps = issue_inputs(i + 1) if i + 1 < NBLK else []
        gath_prev = gath_cur
    acc = drain_and_accum(NBLK - 1, gath_prev, acc)

    acc_v[...] = acc
    pltpu.sync_copy(acc_v, out_hbm.at[wid])


def _phys_view(x):
    s = x.shape
    return (x.reshape(*s[:-2], s[-2] // 8, 8, s[-1] // 128, 128)
            .swapaxes(-2, -3)
            .reshape(-1))


def kernel(z, condition, l, l_target):
    z_phys = _phys_view(z)
    l_phys = _phys_view(l.astype(jnp.int32))
    lt_phys = _phys_view(l_target.astype(jnp.int32))
    cond_phys = _phys_view(condition.astype(jnp.float32))

    mesh = plsc.VectorSubcoreMesh(core_axis_name="c", subcore_axis_name="s")
    fn = pl.kernel(
        _body,
        mesh=mesh,
        out_type=jax.ShapeDtypeStruct((NW, 16), jnp.float32),
        scratch_types=(
            [pltpu.VMEM((BLK,), jnp.int32)] * 2 +
            [pltpu.VMEM((BLK,), jnp.int32)] * 2 +
            [pltpu.VMEM((BLK,), jnp.float32)] * 2 +
            [pltpu.VMEM((BLK,), jnp.int32)] * 2 +
            [pltpu.VMEM((BLK,), jnp.int32)] * 2 +
            [pltpu.VMEM((BLK,), jnp.float32)] * 2 +
            [pltpu.VMEM((BLK,), jnp.float32)] * 2 +
            [pltpu.VMEM((16,), jnp.float32)] +
            [pltpu.SemaphoreType.DMA] * 4
        ),
    )
    partials = fn(z_phys, l_phys, lt_phys, cond_phys)
    return jnp.sum(partials)
